# baseline (speedup 1.0000x reference)
import jax
import jax.numpy as jnp
from jax import lax
from jax.experimental import pallas as pl
from jax.experimental.pallas import tpu as pltpu
from jax.experimental.pallas import tpu_sc as plsc

_BLK = 128
_NBUF = 5


def _make_gather(n_chunks, emb_dim):
    info = plsc.get_sparse_core_info()
    nw = info.num_cores * info.num_subcores
    assert n_chunks % (nw * _NBUF) == 0
    cpw = n_chunks // nw
    niter = cpw // _NBUF
    mesh = plsc.VectorSubcoreMesh(core_axis_name="c", subcore_axis_name="s")

    def body(table_hbm, idx_hbm, out_hbm, idx_v, *bufs):
        rows = bufs[:_NBUF]
        gs = bufs[_NBUF : 2 * _NBUF]
        ws = bufs[2 * _NBUF : 3 * _NBUF]
        wid = lax.axis_index("s") * info.num_cores + lax.axis_index("c")
        chunk0 = wid * cpw
        pltpu.sync_copy(idx_hbm.at[pl.ds(chunk0 * _BLK, cpw * _BLK)], idx_v)

        def gather(j, b):
            pltpu.async_copy(
                table_hbm.at[idx_v.at[pl.ds(j * _BLK, _BLK)]], rows[b], gs[b]
            )

        def wait_gather(j, b):
            pltpu.make_async_copy(
                table_hbm.at[idx_v.at[pl.ds(j * _BLK, _BLK)]], rows[b], gs[b]
            ).wait()

        def wait_wb(b):
            pltpu.make_async_copy(rows[b], out_hbm.at[0], ws[b]).wait()

        for b in range(_NBUF - 1):
            gather(b, b)

        def outer(g, carry):
            for b in range(_NBUF):
                j = g * _NBUF + b
                p = (b - 1) % _NBUF
                wait_gather(j, b)
                pltpu.async_copy(rows[b], out_hbm.at[chunk0 + j], ws[b])
                if b == 0:

                    @pl.when(g >= 1)
                    def _():
                        wait_wb(p)

                    gather(j + _NBUF - 1, p)
                else:

                    @pl.when(g <= niter - 2)
                    def _():
                        wait_wb(p)
                        gather(j + _NBUF - 1, p)

            return carry

        lax.fori_loop(0, niter, outer, 0)
        for b in range(_NBUF):
            wait_wb(b)

    return pl.kernel(
        body,
        out_type=jax.ShapeDtypeStruct((n_chunks, _BLK, emb_dim), jnp.float32),
        mesh=mesh,
        compiler_params=pltpu.CompilerParams(
            use_tc_tiling_on_sc=False, needs_layout_passes=False
        ),
        scratch_types=(
            [pltpu.VMEM((cpw * _BLK,), jnp.int32)]
            + [pltpu.VMEM((_BLK, emb_dim), jnp.float32) for _ in range(_NBUF)]
            + [pltpu.SemaphoreType.DMA for _ in range(2 * _NBUF)]
        ),
    )


def _make_transpose(seq, batch, emb_dim):
    blocks_per_seq = batch // _BLK
    n_chunks = seq * blocks_per_seq
    grp = 8
    half = _BLK // 2
    rows_per_chunk = _BLK * emb_dim // 128

    def body(in_ref, out_ref):
        t = jnp.transpose(in_ref[...])
        for q in range(grp):
            blk = t[:, q * rows_per_chunk : (q + 1) * rows_per_chunk]
            out_ref[0, :, q * _BLK : q * _BLK + half] = blk[:emb_dim]
            out_ref[0, :, q * _BLK + half : (q + 1) * _BLK] = blk[emb_dim:]

    return pl.pallas_call(
        body,
        grid=(n_chunks // grp,),
        in_specs=[
            pl.BlockSpec((grp * rows_per_chunk, 128), lambda c: (c, 0))
        ],
        out_specs=pl.BlockSpec(
            (1, emb_dim, grp * _BLK),
            lambda c: (c // (blocks_per_seq // grp), 0, c % (blocks_per_seq // grp)),
        ),
        out_shape=jax.ShapeDtypeStruct((seq, emb_dim, batch), jnp.float32),
    )


def kernel(x, table):
    b, s = x.shape
    emb_dim = table.shape[1]
    n_chunks = b * s // _BLK
    half = _BLK // 2
    idx_sm = (
        jnp.transpose(x)
        .reshape(s, b // _BLK, 2, half)
        .swapaxes(2, 3)
        .reshape(b * s)
    )
    chunks = _make_gather(n_chunks, emb_dim)(table, idx_sm)
    chunks2d = chunks.reshape(n_chunks * _BLK * emb_dim // 128, 128)
    out_sm = _make_transpose(s, b, emb_dim)(chunks2d)
    return jnp.transpose(out_sm, (2, 0, 1))


# speedup vs baseline: 1.8562x; 1.8562x over previous
import jax
import jax.numpy as jnp
from jax import lax
from jax.experimental import pallas as pl
from jax.experimental.pallas import tpu as pltpu
from jax.experimental.pallas import tpu_sc as plsc

_BLK = 128
_NBUF = 5


def _make_gather(n_chunks, emb_dim, padw, base):
    info = plsc.get_sparse_core_info()
    nw = info.num_cores * info.num_subcores
    assert n_chunks % (nw * _NBUF) == 0
    cpw = n_chunks // nw
    niter = cpw // _NBUF
    mesh = plsc.VectorSubcoreMesh(core_axis_name="c", subcore_axis_name="s")

    half = _BLK // 2
    lanes = 16

    def body(table_hbm, idx_hbm, out_hbm, idx_v, *bufs):
        rows = bufs[:_NBUF]
        idxp = bufs[_NBUF : 2 * _NBUF]
        gs = bufs[2 * _NBUF : 3 * _NBUF]
        ws = bufs[3 * _NBUF : 4 * _NBUF]
        wid = lax.axis_index("s") * info.num_cores + lax.axis_index("c")
        chunk0 = wid * cpw
        pltpu.sync_copy(
            idx_hbm.at[pl.ds((base + chunk0) * _BLK, cpw * _BLK)], idx_v
        )

        il = lax.iota(jnp.int32, lanes)
        pat = (il >> 1) + (il & 1) * half

        def gather(j, b):
            for g in range(_BLK // lanes):
                idxp[b][pl.ds(g * lanes, lanes)] = plsc.load_gather(
                    idx_v, [j * _BLK + (g * (lanes // 2)) + pat]
                )
            pltpu.async_copy(table_hbm.at[idxp[b]], rows[b], gs[b])

        def wait_gather(j, b):
            pltpu.make_async_copy(
                table_hbm.at[idxp[b]], rows[b], gs[b]
            ).wait()

        def wait_wb(b):
            pltpu.make_async_copy(
                rows[b].at[:, pl.ds(0, emb_dim)], out_hbm.at[0], ws[b]
            ).wait()

        for b in range(_NBUF - 1):
            gather(b, b)

        def outer(g, carry):
            for b in range(_NBUF):
                j = g * _NBUF + b
                p = (b - 1) % _NBUF
                wait_gather(j, b)
                pltpu.async_copy(
                    rows[b].at[:, pl.ds(0, emb_dim)], out_hbm.at[chunk0 + j], ws[b]
                )
                if b == 0:

                    @pl.when(g >= 1)
                    def _():
                        wait_wb(p)

                    gather(j + _NBUF - 1, p)
                else:

                    @pl.when(g <= niter - 2)
                    def _():
                        wait_wb(p)
                        gather(j + _NBUF - 1, p)

            return carry

        lax.fori_loop(0, niter, outer, 0)
        for b in range(_NBUF):
            wait_wb(b)

    return pl.kernel(
        body,
        out_type=jax.ShapeDtypeStruct((n_chunks, _BLK, emb_dim), jnp.float32),
        mesh=mesh,
        compiler_params=pltpu.CompilerParams(
            use_tc_tiling_on_sc=False, needs_layout_passes=False
        ),
        scratch_types=(
            [pltpu.VMEM((cpw * _BLK,), jnp.int32)]
            + [pltpu.VMEM((_BLK, padw), jnp.float32) for _ in range(_NBUF)]
            + [pltpu.VMEM((_BLK,), jnp.int32) for _ in range(_NBUF)]
            + [pltpu.SemaphoreType.DMA for _ in range(2 * _NBUF)]
        ),
    )


def _make_table_transpose(n_tokens, emb_dim, padw, blk):
    grid = -(-n_tokens // blk)

    def body(in_ref, out_ref):
        out_ref[:, :emb_dim] = jnp.transpose(in_ref[...])

    return pl.pallas_call(
        body,
        grid=(grid,),
        in_specs=[pl.BlockSpec((emb_dim, blk), lambda m: (0, m))],
        out_specs=pl.BlockSpec((blk, padw), lambda m: (m, 0)),
        out_shape=jax.ShapeDtypeStruct((grid * blk, padw), jnp.float32),
    )


def _make_transpose(seq, batch, emb_dim, s_base, s_count, aliased):
    blocks_per_seq = batch // _BLK
    grp = 32
    half = _BLK // 2
    rows_per_chunk = _BLK * emb_dim // 128

    def body(in_ref, *refs):
        out_ref = refs[-1]
        t = jnp.transpose(in_ref[...])
        for q in range(grp):
            blk = t[:, q * rows_per_chunk : (q + 1) * rows_per_chunk]
            out_ref[0, :, q * _BLK : q * _BLK + half] = blk[:emb_dim]
            out_ref[0, :, q * _BLK + half : (q + 1) * _BLK] = blk[emb_dim:]

    in_specs = [pl.BlockSpec((grp * rows_per_chunk, 128), lambda c: (c, 0))]
    kwargs = {}
    if aliased:
        in_specs.append(pl.BlockSpec(memory_space=pl.ANY))
        kwargs["input_output_aliases"] = {1: 0}
    return pl.pallas_call(
        body,
        grid=(s_count * blocks_per_seq // grp,),
        in_specs=in_specs,
        out_specs=pl.BlockSpec(
            (1, emb_dim, grp * _BLK),
            lambda c: (
                s_base + c // (blocks_per_seq // grp),
                0,
                c % (blocks_per_seq // grp),
            ),
        ),
        out_shape=jax.ShapeDtypeStruct((seq, emb_dim, batch), jnp.float32),
        **kwargs,
    )


def kernel(x, table):
    b, s = x.shape
    emb_dim = table.shape[1]
    n_chunks = b * s // _BLK
    idx_sm = jnp.transpose(x).reshape(b * s)
    tableg = _make_table_transpose(table.shape[0], emb_dim, 128, 4096)(
        jnp.transpose(table)
    )
    k_parts = 2
    nh = n_chunks // k_parts
    rows_h = nh * _BLK * emb_dim // 128
    sh = s // k_parts
    gathers = [
        _make_gather(nh, emb_dim, 128, k * nh)(tableg, idx_sm)
        for k in range(k_parts)
    ]
    out_sm = _make_transpose(s, b, emb_dim, 0, sh, False)(
        gathers[0].reshape(rows_h, 128)
    )
    for k in range(1, k_parts):
        out_sm = _make_transpose(s, b, emb_dim, k * sh, sh, True)(
            gathers[k].reshape(rows_h, 128), out_sm
        )
    return jnp.transpose(out_sm, (2, 0, 1))
